# baseline (speedup 1.0000x reference)
import functools

import jax
import jax.numpy as jnp
from jax import lax
from jax.experimental import pallas as pl
from jax.experimental.pallas import tpu as pltpu
from jax.experimental.pallas import tpu_sc as plsc

N_NODES = 10000
N_EDGES = 320000
D = 128
MAX_Z = 16

NC = 2
NS = 16
NW = NC * NS
EDGES_PER_TILE = N_EDGES // NW
K = 125
NCHUNK = EDGES_PER_TILE // K
ZR = 80
NZCH = N_NODES // ZR

BLK = 1000
GRID = N_NODES // BLK


def _norm(x):
  return jnp.clip(jnp.sqrt(jnp.sum(x * x, axis=-1, keepdims=True)), 1e-15, None)


def _artanh(x):
  x = jnp.clip(x, -1.0 + 1e-7, 1.0 - 1e-7)
  return 0.5 * (jnp.log1p(x) - jnp.log1p(-x))


def _proj(x):
  n = _norm(x)
  maxnorm = 1.0 - 1e-5
  return jnp.where(n > maxnorm, x / n * maxnorm, x)


def _expmap0(u):
  n = _norm(u)
  return _proj(jnp.tanh(n) * u / n)


def _logmap0(p):
  n = _norm(p)
  return _artanh(n) * p / n


def _mobius_add(x, y):
  x2 = jnp.sum(x * x, axis=-1, keepdims=True)
  y2 = jnp.sum(y * y, axis=-1, keepdims=True)
  xy = jnp.sum(x * y, axis=-1, keepdims=True)
  num = (1 + 2 * xy + y2) * x + (1 - x2) * y
  denom = 1 + 2 * xy + x2 * y2
  return num / jnp.clip(denom, 1e-15, None)


def _stage_in_body(h_ref, wt_ref, b_ref, d_ref, m_ref, o_ref, w_ref):
  x = _proj(_expmap0(h_ref[...]))
  mv = _expmap0(
      jnp.dot(_logmap0(x), wt_ref[...], preferred_element_type=jnp.float32))
  h1 = _proj(_mobius_add(mv, _expmap0(b_ref[...])))
  o_ref[...] = _logmap0(h1)
  w_ref[...] = jnp.exp(-d_ref[...]) * m_ref[...]


def _stage_mid_body(p_ref, nm_ref, wt_ref, b_ref, o_ref):
  agg = (p_ref[0] + p_ref[1]) * nm_ref[...]
  h2 = _proj(_expmap0(agg))
  h3 = _proj(_expmap0(jax.nn.relu(_logmap0(h2))))
  mv = _expmap0(
      jnp.dot(_logmap0(h3), wt_ref[...], preferred_element_type=jnp.float32))
  h1 = _proj(_mobius_add(mv, _expmap0(b_ref[...])))
  o_ref[...] = _logmap0(h1)


def _stage_out_body(p_ref, nm_ref, wt_ref, b_ref, o_ref):
  agg = (p_ref[0] + p_ref[1]) * nm_ref[...]
  h2 = _proj(_expmap0(agg))
  h3 = _proj(_expmap0(jax.nn.relu(_logmap0(h2))))
  out_t = _logmap0(h3)
  o_ref[...] = (
      jnp.dot(out_t, wt_ref[...], preferred_element_type=jnp.float32)
      + b_ref[...])


_full = pl.BlockSpec((D, D), lambda i: (0, 0))
_row = pl.BlockSpec((BLK, D), lambda i: (i, 0))
_rowmask = pl.BlockSpec((BLK, 1), lambda i: (i, 0))
_bias = pl.BlockSpec((1, D), lambda i: (0, 0))
_pairs = pl.BlockSpec((2, BLK, D), lambda i: (0, i, 0))
_EROWS = N_EDGES // D // GRID
_edgeblk = pl.BlockSpec((1, _EROWS, D), lambda i: (i, 0, 0))

_stage_in = pl.pallas_call(
    _stage_in_body,
    grid=(GRID,),
    in_specs=[_row, _full, _bias, _edgeblk, _edgeblk],
    out_specs=[_row, _edgeblk],
    out_shape=[
        jax.ShapeDtypeStruct((N_NODES, D), jnp.float32),
        jax.ShapeDtypeStruct((GRID, _EROWS, D), jnp.float32),
    ],
)

_stage_mid = pl.pallas_call(
    _stage_mid_body,
    grid=(GRID,),
    in_specs=[_pairs, _rowmask, _full, _bias],
    out_specs=_row,
    out_shape=jax.ShapeDtypeStruct((N_NODES, D), jnp.float32),
)

_stage_out = pl.pallas_call(
    _stage_out_body,
    grid=(GRID,),
    in_specs=[
        _pairs, _rowmask,
        pl.BlockSpec((D, MAX_Z), lambda i: (0, 0)),
        pl.BlockSpec((1, MAX_Z), lambda i: (0, 0)),
    ],
    out_specs=pl.BlockSpec((BLK, MAX_Z), lambda i: (i, 0)),
    out_shape=jax.ShapeDtypeStruct((N_NODES, MAX_Z), jnp.float32),
)


@functools.lru_cache(maxsize=1)
def _get_sc_agg():
  @functools.partial(
      pl.kernel,
      out_type=jax.ShapeDtypeStruct((NC, N_NODES, D), jnp.float32),
      mesh=plsc.VectorSubcoreMesh(core_axis_name="c", subcore_axis_name="s"),
      scratch_types=[
          pltpu.VMEM((3, 1, K), jnp.int32),
          pltpu.VMEM((3, 1, K), jnp.int32),
          pltpu.VMEM((3, 1, K), jnp.float32),
          pltpu.VMEM((3, K, D), jnp.float32),
          pltpu.VMEM_SHARED((N_NODES, D), jnp.float32),
          pltpu.SemaphoreType.DMA,
          pltpu.SemaphoreType.DMA,
          pltpu.SemaphoreType.DMA,
          pltpu.SemaphoreType.DMA,
          pltpu.SemaphoreType.DMA,
          pltpu.SemaphoreType.DMA,
          pltpu.SemaphoreType.DMA,
          pltpu.SemaphoreType.DMA,
          pltpu.SemaphoreType.DMA,
      ],
  )
  def _sc_agg(xt_hbm, edges_hbm, w_hbm, out_hbm,
              src_v, dst_v, w_v, rows_v, acc_sh,
              gsem0, gsem1, gsem2,
              ssem0, ssem1, ssem2,
              isem0, isem1, isem2):
    cid = lax.axis_index("c")
    sid = lax.axis_index("s")
    wid = cid * NS + sid
    zero16 = jnp.zeros((16,), jnp.float32)
    gsem = (gsem0, gsem1, gsem2)
    ssem = (ssem0, ssem1, ssem2)
    isem = (isem0, isem1, isem2)

    def _zrow(r, carry):
      for c in range(D // 16):
        rows_v[0, r, pl.ds(c * 16, 16)] = zero16
      return carry

    lax.fori_loop(0, ZR, _zrow, 0)
    for t in range((NZCH + NS - 1) // NS):
      ch = sid + t * NS

      @pl.when(ch < NZCH)
      def _():
        pltpu.sync_copy(rows_v.at[0, pl.ds(0, ZR)],
                        acc_sh.at[pl.ds(ch * ZR, ZR)])

    plsc.subcore_barrier()

    def _fire_idx(i, q):
      pltpu.async_copy(edges_hbm.at[0, wid, i], src_v.at[q], isem[q])
      pltpu.async_copy(edges_hbm.at[1, wid, i], dst_v.at[q], isem[q])
      pltpu.async_copy(w_hbm.at[wid, i], w_v.at[q], isem[q])

    def _wait_idx(i, q):
      pltpu.make_async_copy(edges_hbm.at[0, wid, i], src_v.at[q],
                            isem[q]).wait()
      pltpu.make_async_copy(edges_hbm.at[1, wid, i], dst_v.at[q],
                            isem[q]).wait()
      pltpu.make_async_copy(w_hbm.at[wid, i], w_v.at[q], isem[q]).wait()

    def _start_gather(q, b):
      pltpu.async_copy(xt_hbm.at[src_v.at[q, 0]], rows_v.at[b], gsem[b])

    def _wait_gather(q, b):
      pltpu.make_async_copy(xt_hbm.at[src_v.at[q, 0]], rows_v.at[b],
                            gsem[b]).wait()

    def _start_scatter(q, b):
      pltpu.async_copy(rows_v.at[b], acc_sh.at[dst_v.at[q, 0]], ssem[b],
                       add=True)

    def _wait_scatter(q, b):
      pltpu.make_async_copy(rows_v.at[b], acc_sh.at[dst_v.at[q, 0]],
                            ssem[b]).wait()

    def _scale(q, b):
      ngrp = K // 16
      tail = K - ngrp * 16

      def _grp(g, c2):
        wvec = w_v[q, 0, pl.ds(g * 16, 16)]
        for l in range(16):
          wsplat = jnp.full((16,), wvec[l], jnp.float32)
          j = g * 16 + l
          for c in range(D // 16):
            sl = (b, j, pl.ds(c * 16, 16))
            rows_v[sl] = rows_v[sl] * wsplat
        return c2

      lax.fori_loop(0, ngrp, _grp, 0)
      if tail:
        wvec = w_v[q, 0, pl.ds(K - 16, 16)]
        for l in range(tail):
          wsplat = jnp.full((16,), wvec[16 - tail + l], jnp.float32)
          j = ngrp * 16 + l
          for c in range(D // 16):
            sl = (b, j, pl.ds(c * 16, 16))
            rows_v[sl] = rows_v[sl] * wsplat

    def _step(i, u, guard_first, fire_ok, scale_ok):
      q1 = (u + 1) % 3
      q2 = (u + 2) % 3

      if guard_first:
        @pl.when(i > 0)
        def _():
          _wait_scatter(q2, q2)
      elif i > 0:
        _wait_scatter(q2, q2)
      _start_scatter(u, u)
      if fire_ok:
        _fire_idx(i + 2, q2)
      if scale_ok:
        _wait_gather(q1, q1)
        _scale(q1, q1)
      if fire_ok:
        _wait_idx(i + 2, q2)
        _start_gather(q2, q2)

    _fire_idx(0, 0)
    _fire_idx(1, 1)
    _wait_idx(0, 0)
    _start_gather(0, 0)
    _wait_idx(1, 1)
    _start_gather(1, 1)
    _wait_gather(0, 0)
    _scale(0, 0)

    def _triple(t, carry):
      for u in range(3):
        _step(3 * t + u, u, True, True, True)
      return carry

    lax.fori_loop(0, (NCHUNK - 2) // 3, _triple, 0)

    _step(NCHUNK - 2, (NCHUNK - 2) % 3, False, False, True)
    _step(NCHUNK - 1, (NCHUNK - 1) % 3, False, False, False)
    _wait_scatter((NCHUNK - 1) % 3, (NCHUNK - 1) % 3)
    plsc.subcore_barrier()

    for t in range((NZCH + NS - 1) // NS):
      ch = sid + t * NS

      @pl.when(ch < NZCH)
      def _():
        sl = pl.ds(ch * ZR, ZR)
        pltpu.sync_copy(acc_sh.at[sl], out_hbm.at[cid, sl])

  return _sc_agg


def kernel(h, distances, edges, node_mask, edge_mask, W1, b1, W2, b2,
           W_out, b_out):
  edges4 = edges.astype(jnp.int32).reshape(2, NW, NCHUNK, 1, K)

  sc_agg = _get_sc_agg()
  xt1, w = _stage_in(h, W1.T, b1.reshape(1, D),
                     distances.reshape(GRID, _EROWS, D),
                     edge_mask.reshape(GRID, _EROWS, D))
  w = w.reshape(NW, NCHUNK, 1, K)
  p1 = sc_agg(xt1, edges4, w)
  xt2 = _stage_mid(p1, node_mask, W2.T, b2.reshape(1, D))
  p2 = sc_agg(xt2, edges4, w)
  return _stage_out(p2, node_mask, W_out.T, b_out.reshape(1, MAX_Z))


# speedup vs baseline: 1.1820x; 1.1820x over previous
import functools

import jax
import jax.numpy as jnp
from jax import lax
from jax.experimental import pallas as pl
from jax.experimental.pallas import tpu as pltpu
from jax.experimental.pallas import tpu_sc as plsc

N_NODES = 10000
N_EDGES = 320000
D = 128
MAX_Z = 16

NC = 2
NS = 16
NW = NC * NS
EDGES_PER_TILE = N_EDGES // NW
K = 125
NCHUNK = EDGES_PER_TILE // K
ZR = 80
NZCH = N_NODES // ZR

BLK = 1000
GRID = N_NODES // BLK


def _norm(x):
  return jnp.clip(jnp.sqrt(jnp.sum(x * x, axis=-1, keepdims=True)), 1e-15, None)


def _artanh(x):
  x = jnp.clip(x, -1.0 + 1e-7, 1.0 - 1e-7)
  return 0.5 * (jnp.log1p(x) - jnp.log1p(-x))


def _proj(x):
  n = _norm(x)
  maxnorm = 1.0 - 1e-5
  return jnp.where(n > maxnorm, x / n * maxnorm, x)


def _expmap0(u):
  n = _norm(u)
  return _proj(jnp.tanh(n) * u / n)


def _logmap0(p):
  n = _norm(p)
  return _artanh(n) * p / n


def _mobius_add(x, y):
  x2 = jnp.sum(x * x, axis=-1, keepdims=True)
  y2 = jnp.sum(y * y, axis=-1, keepdims=True)
  xy = jnp.sum(x * y, axis=-1, keepdims=True)
  num = (1 + 2 * xy + y2) * x + (1 - x2) * y
  denom = 1 + 2 * xy + x2 * y2
  return num / jnp.clip(denom, 1e-15, None)


def _stage_in_body(h_ref, wt_ref, b_ref, d_ref, m_ref, o_ref, w_ref):
  x = _proj(_expmap0(h_ref[...]))
  mv = _expmap0(
      jnp.dot(_logmap0(x), wt_ref[...], preferred_element_type=jnp.float32))
  h1 = _proj(_mobius_add(mv, _expmap0(b_ref[...])))
  o_ref[...] = _logmap0(h1)
  w_ref[...] = jnp.exp(-d_ref[...]) * m_ref[...]


def _stage_mid_body(p_ref, nm_ref, wt_ref, b_ref, o_ref):
  agg = (p_ref[0] + p_ref[1]) * nm_ref[...]
  h2 = _proj(_expmap0(agg))
  h3 = _proj(_expmap0(jax.nn.relu(_logmap0(h2))))
  mv = _expmap0(
      jnp.dot(_logmap0(h3), wt_ref[...], preferred_element_type=jnp.float32))
  h1 = _proj(_mobius_add(mv, _expmap0(b_ref[...])))
  o_ref[...] = _logmap0(h1)


def _stage_out_body(p_ref, nm_ref, wt_ref, b_ref, o_ref):
  agg = (p_ref[0] + p_ref[1]) * nm_ref[...]
  h2 = _proj(_expmap0(agg))
  h3 = _proj(_expmap0(jax.nn.relu(_logmap0(h2))))
  out_t = _logmap0(h3)
  o_ref[...] = (
      jnp.dot(out_t, wt_ref[...], preferred_element_type=jnp.float32)
      + b_ref[...])


_full = pl.BlockSpec((D, D), lambda i: (0, 0))
_row = pl.BlockSpec((BLK, D), lambda i: (i, 0))
_rowmask = pl.BlockSpec((BLK, 1), lambda i: (i, 0))
_bias = pl.BlockSpec((1, D), lambda i: (0, 0))
_pairs = pl.BlockSpec((2, BLK, D), lambda i: (0, i, 0))
_EROWS = N_EDGES // D // GRID
_edgeblk = pl.BlockSpec((1, _EROWS, D), lambda i: (i, 0, 0))

_stage_in = pl.pallas_call(
    _stage_in_body,
    grid=(GRID,),
    in_specs=[_row, _full, _bias, _edgeblk, _edgeblk],
    out_specs=[_row, _edgeblk],
    out_shape=[
        jax.ShapeDtypeStruct((N_NODES, D), jnp.float32),
        jax.ShapeDtypeStruct((GRID, _EROWS, D), jnp.float32),
    ],
)

_stage_mid = pl.pallas_call(
    _stage_mid_body,
    grid=(GRID,),
    in_specs=[_pairs, _rowmask, _full, _bias],
    out_specs=_row,
    out_shape=jax.ShapeDtypeStruct((N_NODES, D), jnp.float32),
)

_stage_out = pl.pallas_call(
    _stage_out_body,
    grid=(GRID,),
    in_specs=[
        _pairs, _rowmask,
        pl.BlockSpec((D, MAX_Z), lambda i: (0, 0)),
        pl.BlockSpec((1, MAX_Z), lambda i: (0, 0)),
    ],
    out_specs=pl.BlockSpec((BLK, MAX_Z), lambda i: (i, 0)),
    out_shape=jax.ShapeDtypeStruct((N_NODES, MAX_Z), jnp.float32),
)


@functools.lru_cache(maxsize=1)
def _get_sc_agg():
  @functools.partial(
      pl.kernel,
      out_type=jax.ShapeDtypeStruct((NC, N_NODES, D), jnp.float32),
      mesh=plsc.VectorSubcoreMesh(core_axis_name="c", subcore_axis_name="s"),
      scratch_types=[
          pltpu.VMEM((4, 1, K), jnp.int32),
          pltpu.VMEM((4, 1, K), jnp.int32),
          pltpu.VMEM((4, 1, K), jnp.float32),
          pltpu.VMEM((2, K, D), jnp.float32),
          pltpu.VMEM_SHARED((N_NODES, D), jnp.float32),
          pltpu.SemaphoreType.DMA,
          pltpu.SemaphoreType.DMA,
          pltpu.SemaphoreType.DMA,
          pltpu.SemaphoreType.DMA,
          pltpu.SemaphoreType.DMA,
          pltpu.SemaphoreType.DMA,
          pltpu.SemaphoreType.DMA,
          pltpu.SemaphoreType.DMA,
          pltpu.SemaphoreType.DMA,
          pltpu.SemaphoreType.DMA,
          pltpu.SemaphoreType.DMA,
          pltpu.SemaphoreType.DMA,
      ],
  )
  def _sc_agg(xt_hbm, edges_hbm, w_hbm, out_hbm,
              src_v, dst_v, w_v, rows_v, acc_sh,
              gsem0, gsem1, gsem2, gsem3,
              ssem0, ssem1, ssem2, ssem3,
              isem0, isem1, isem2, isem3):
    cid = lax.axis_index("c")
    sid = lax.axis_index("s")
    wid = cid * NS + sid
    zero16 = jnp.zeros((16,), jnp.float32)
    gsem = (gsem0, gsem1, gsem2, gsem3)
    ssem = (ssem0, ssem1, ssem2, ssem3)
    isem = (isem0, isem1, isem2, isem3)

    def _zrow(r, carry):
      for c in range(D // 16):
        rows_v[0, r, pl.ds(c * 16, 16)] = zero16
      return carry

    lax.fori_loop(0, ZR, _zrow, 0)
    for t in range((NZCH + NS - 1) // NS):
      ch = sid + t * NS

      @pl.when(ch < NZCH)
      def _():
        pltpu.sync_copy(rows_v.at[0, pl.ds(0, ZR)],
                        acc_sh.at[pl.ds(ch * ZR, ZR)])

    plsc.subcore_barrier()

    def _fire_idx(i, q):
      pltpu.async_copy(edges_hbm.at[0, wid, i], src_v.at[q], isem[q])
      pltpu.async_copy(edges_hbm.at[1, wid, i], dst_v.at[q], isem[q])
      pltpu.async_copy(w_hbm.at[wid, i], w_v.at[q], isem[q])

    def _wait_idx(i, q):
      pltpu.make_async_copy(edges_hbm.at[0, wid, i], src_v.at[q],
                            isem[q]).wait()
      pltpu.make_async_copy(edges_hbm.at[1, wid, i], dst_v.at[q],
                            isem[q]).wait()
      pltpu.make_async_copy(w_hbm.at[wid, i], w_v.at[q], isem[q]).wait()

    def _start_gather(q, b):
      pltpu.async_copy(xt_hbm.at[src_v.at[q, 0]], rows_v.at[b], gsem[b])

    def _wait_gather(q, b):
      pltpu.make_async_copy(xt_hbm.at[src_v.at[q, 0]], rows_v.at[b],
                            gsem[b]).wait()

    def _start_scatter(q, b):
      pltpu.async_copy(rows_v.at[b], acc_sh.at[dst_v.at[q, 0]], ssem[b],
                       add=True)

    def _wait_scatter(q, b):
      pltpu.make_async_copy(rows_v.at[b], acc_sh.at[dst_v.at[q, 0]],
                            ssem[b]).wait()

    def _scale(q, b):
      ngrp = K // 16
      tail = K - ngrp * 16

      def _grp(g, c2):
        wvec = w_v[q, 0, pl.ds(g * 16, 16)]
        for l in range(16):
          wsplat = jnp.full((16,), wvec[l], jnp.float32)
          j = g * 16 + l
          for c in range(D // 16):
            sl = (b, j, pl.ds(c * 16, 16))
            rows_v[sl] = rows_v[sl] * wsplat
        return c2

      lax.fori_loop(0, ngrp, _grp, 0)
      if tail:
        wvec = w_v[q, 0, pl.ds(K - 16, 16)]
        for l in range(tail):
          wsplat = jnp.full((16,), wvec[16 - tail + l], jnp.float32)
          j = ngrp * 16 + l
          for c in range(D // 16):
            sl = (b, j, pl.ds(c * 16, 16))
            rows_v[sl] = rows_v[sl] * wsplat

    def _step(i, u, guard_first, fire_ok, next_ok):
      b = u % 2
      nb = 1 - b
      q = u
      qn = (u + 1) % 4
      qp = (u + 3) % 4

      _wait_gather(q, b)
      if guard_first:
        @pl.when(i > 0)
        def _():
          _wait_scatter(qp, nb)
      else:
        if i > 0:
          _wait_scatter(qp, nb)
      if fire_ok:
        _fire_idx(i + 2, (u + 2) % 4)
      if next_ok:
        _wait_idx(i + 1, qn)
        _start_gather(qn, nb)
      _scale(q, b)
      _start_scatter(q, b)

    _fire_idx(0, 0)
    _wait_idx(0, 0)
    _start_gather(0, 0)
    _fire_idx(1, 1)

    def _quad(t, carry):
      for u in range(4):
        _step(4 * t + u, u, True, True, True)
      return carry

    lax.fori_loop(0, NCHUNK // 4 - 1, _quad, 0)

    for u in range(4):
      i = NCHUNK - 4 + u
      _step(i, u, False, i + 2 < NCHUNK, i + 1 < NCHUNK)
    _wait_scatter((NCHUNK - 1) % 4, (NCHUNK - 1) % 2)
    plsc.subcore_barrier()

    for t in range((NZCH + NS - 1) // NS):
      ch = sid + t * NS

      @pl.when(ch < NZCH)
      def _():
        sl = pl.ds(ch * ZR, ZR)
        pltpu.sync_copy(acc_sh.at[sl], out_hbm.at[cid, sl])

  return _sc_agg


def kernel(h, distances, edges, node_mask, edge_mask, W1, b1, W2, b2,
           W_out, b_out):
  edges4 = edges.astype(jnp.int32).reshape(2, NW, NCHUNK, 1, K)

  sc_agg = _get_sc_agg()
  xt1, w = _stage_in(h, W1.T, b1.reshape(1, D),
                     distances.reshape(GRID, _EROWS, D),
                     edge_mask.reshape(GRID, _EROWS, D))
  w = w.reshape(NW, NCHUNK, 1, K)
  p1 = sc_agg(xt1, edges4, w)
  xt2 = _stage_mid(p1, node_mask, W2.T, b2.reshape(1, D))
  p2 = sc_agg(xt2, edges4, w)
  return _stage_out(p2, node_mask, W_out.T, b_out.reshape(1, MAX_Z))
